# baseline (speedup 1.0000x reference)
import functools

import jax
import jax.numpy as jnp
from jax import lax
from jax.experimental import pallas as pl
from jax.experimental.pallas import tpu as pltpu
from jax.experimental.pallas import tpu_sc as plsc

B = 16
NB = 4


def _sc_lookup(camindex, dataset_type, wcam1f, bcam1f, wcam2f, bcam2f):
    mesh = plsc.VectorSubcoreMesh(core_axis_name="c", subcore_axis_name="s")

    @functools.partial(
        pl.kernel,
        mesh=mesh,
        compiler_params=pltpu.CompilerParams(needs_layout_passes=False),
        out_type=jax.ShapeDtypeStruct((6, B), jnp.float32),
        scratch_types=[
            pltpu.VMEM((B,), jnp.int32),
            pltpu.VMEM((B,), jnp.int32),
            pltpu.VMEM((300,), jnp.float32),
            pltpu.VMEM((300,), jnp.float32),
            pltpu.VMEM((150,), jnp.float32),
            pltpu.VMEM((150,), jnp.float32),
            pltpu.VMEM((6, B), jnp.float32),
            pltpu.SemaphoreType.DMA,
        ],
    )
    def lookup(cam_h, dt_h, wc1_h, bc1_h, wc2_h, bc2_h, wb_out,
               cam_v, dt_v, wc1_v, bc1_v, wc2_v, bc2_v, wb_v, sem):
        wid = lax.axis_index("s") * 2 + lax.axis_index("c")

        @pl.when(wid == 0)
        def _():
            copies = [
                pltpu.async_copy(cam_h, cam_v, sem),
                pltpu.async_copy(dt_h, dt_v, sem),
                pltpu.async_copy(wc1_h, wc1_v, sem),
                pltpu.async_copy(bc1_h, bc1_v, sem),
                pltpu.async_copy(wc2_h, wc2_v, sem),
                pltpu.async_copy(bc2_h, bc2_v, sem),
            ]
            for cp in copies:
                cp.wait()
            cam3 = cam_v[...] * 3
            use1 = dt_v[...] == 0
            for c in range(3):
                wb_v[c, :] = jnp.where(
                    use1,
                    plsc.load_gather(wc1_v, [cam3 + c]),
                    plsc.load_gather(wc2_v, [cam3 + c]))
                wb_v[3 + c, :] = jnp.where(
                    use1,
                    plsc.load_gather(bc1_v, [cam3 + c]),
                    plsc.load_gather(bc2_v, [cam3 + c]))
            pltpu.sync_copy(wb_v, wb_out)

    return lookup(camindex, dataset_type, wcam1f, bcam1f, wcam2f, bcam2f)


def _affine_body(idr_ref, dtr_ref, wbcam_ref, *refs):
    wi1_refs = refs[0 * NB:1 * NB]
    bi1_refs = refs[1 * NB:2 * NB]
    wi2_refs = refs[2 * NB:3 * NB]
    bi2_refs = refs[3 * NB:4 * NB]
    img_ref = refs[4 * NB]
    out_ref = refs[4 * NB + 1]
    b_i = pl.program_id(0)
    for j in range(NB):
        s = b_i * NB + j
        use1 = dtr_ref[s] == 0
        for c in range(3):
            wi = jnp.where(use1, wi1_refs[j][0, 0, c], wi2_refs[j][0, 0, c])
            bi = jnp.where(use1, bi1_refs[j][0, 0, c], bi2_refs[j][0, 0, c])
            w = wbcam_ref[c, s] + wi
            bb = wbcam_ref[3 + c, s] + bi
            out_ref[j, c] = img_ref[j, c] * w + bb


def _tc_affine(idindex, dataset_type, wbcam,
               wident1, bident1, wident2, bident2, image):
    def row_map(j):
        return lambda bi, idr, dtr: (idr[bi * NB + j], 0, 0)

    row_specs = [pl.BlockSpec((1, 1, 3), row_map(j)) for j in range(NB)]
    grid_spec = pltpu.PrefetchScalarGridSpec(
        num_scalar_prefetch=2,
        grid=(B // NB,),
        in_specs=[
            pl.BlockSpec(memory_space=pltpu.SMEM),
        ] + row_specs * 4 + [
            pl.BlockSpec((NB, 3, 512, 512),
                         lambda bi, idr, dtr: (bi, 0, 0, 0)),
        ],
        out_specs=pl.BlockSpec((NB, 3, 512, 512),
                               lambda bi, idr, dtr: (bi, 0, 0, 0)),
    )
    wi1 = wident1.reshape(-1, 1, 3)
    bi1 = bident1.reshape(-1, 1, 3)
    wi2 = wident2.reshape(-1, 1, 3)
    bi2 = bident2.reshape(-1, 1, 3)
    return pl.pallas_call(
        _affine_body,
        grid_spec=grid_spec,
        out_shape=jax.ShapeDtypeStruct(image.shape, image.dtype),
        compiler_params=pltpu.CompilerParams(
            dimension_semantics=("arbitrary",)),
    )(idindex, dataset_type, wbcam,
      *([wi1] * NB), *([bi1] * NB), *([wi2] * NB), *([bi2] * NB), image)


@jax.jit
def kernel(image, camindex, idindex, dataset_type,
           wcam1, bcam1, wident1, bident1,
           wcam2, bcam2, wident2, bident2):
    wbcam = _sc_lookup(camindex, dataset_type,
                       wcam1.reshape(-1), bcam1.reshape(-1),
                       wcam2.reshape(-1), bcam2.reshape(-1))
    return _tc_affine(idindex, dataset_type, wbcam,
                      wident1, bident1, wident2, bident2, image)


# speedup vs baseline: 1.2939x; 1.2939x over previous
import functools

import jax
import jax.numpy as jnp
from jax import lax
from jax.experimental import pallas as pl
from jax.experimental.pallas import tpu as pltpu
from jax.experimental.pallas import tpu_sc as plsc

B = 16
NB = 4


def _sc_lookup(camindex, dataset_type, wcam1f, bcam1f, wcam2f, bcam2f):
    mesh = plsc.VectorSubcoreMesh(core_axis_name="c", subcore_axis_name="s")

    @functools.partial(
        pl.kernel,
        mesh=mesh,
        compiler_params=pltpu.CompilerParams(needs_layout_passes=False),
        out_type=jax.ShapeDtypeStruct((6, B), jnp.float32),
        scratch_types=[
            pltpu.VMEM((B,), jnp.int32),
            pltpu.VMEM((B,), jnp.int32),
            pltpu.VMEM((300,), jnp.float32),
            pltpu.VMEM((300,), jnp.float32),
            pltpu.VMEM((150,), jnp.float32),
            pltpu.VMEM((150,), jnp.float32),
            pltpu.VMEM((6, B), jnp.float32),
            pltpu.SemaphoreType.DMA,
        ],
    )
    def lookup(cam_h, dt_h, wc1_h, bc1_h, wc2_h, bc2_h, wb_out,
               cam_v, dt_v, wc1_v, bc1_v, wc2_v, bc2_v, wb_v, sem):
        wid = lax.axis_index("s") * 2 + lax.axis_index("c")

        @pl.when(wid == 0)
        def _():
            copies = [
                pltpu.async_copy(cam_h, cam_v, sem),
                pltpu.async_copy(dt_h, dt_v, sem),
                pltpu.async_copy(wc1_h, wc1_v, sem),
                pltpu.async_copy(bc1_h, bc1_v, sem),
                pltpu.async_copy(wc2_h, wc2_v, sem),
                pltpu.async_copy(bc2_h, bc2_v, sem),
            ]
            for cp in copies:
                cp.wait()
            cam3 = cam_v[...] * 3
            use1 = dt_v[...] == 0
            for c in range(3):
                wb_v[c, :] = jnp.where(
                    use1,
                    plsc.load_gather(wc1_v, [cam3 + c]),
                    plsc.load_gather(wc2_v, [cam3 + c]))
                wb_v[3 + c, :] = jnp.where(
                    use1,
                    plsc.load_gather(bc1_v, [cam3 + c]),
                    plsc.load_gather(bc2_v, [cam3 + c]))
            pltpu.sync_copy(wb_v, wb_out)

    return lookup(camindex, dataset_type, wcam1f, bcam1f, wcam2f, bcam2f)


def _affine_body(idr_ref, dtr_ref, wbcam_ref, *refs):
    wi1_refs = refs[0 * NB:1 * NB]
    bi1_refs = refs[1 * NB:2 * NB]
    wi2_refs = refs[2 * NB:3 * NB]
    bi2_refs = refs[3 * NB:4 * NB]
    img_ref = refs[4 * NB]
    out_ref = refs[4 * NB + 1]
    b_i = pl.program_id(0)
    for j in range(NB):
        s = b_i * NB + j
        use1 = dtr_ref[s] == 0
        for c in range(3):
            wi = jnp.where(use1, wi1_refs[j][0, 0, c], wi2_refs[j][0, 0, c])
            bi = jnp.where(use1, bi1_refs[j][0, 0, c], bi2_refs[j][0, 0, c])
            w = wbcam_ref[c, s] + wi
            bb = wbcam_ref[3 + c, s] + bi
            out_ref[j, c] = img_ref[j, c] * w + bb


def _tc_affine(idindex, dataset_type, wbcam,
               wident1, bident1, wident2, bident2, image):
    def row_map(j):
        return lambda bi, idr, dtr: (idr[bi * NB + j], 0, 0)

    row_specs = [pl.BlockSpec((1, 1, 3), row_map(j)) for j in range(NB)]
    grid_spec = pltpu.PrefetchScalarGridSpec(
        num_scalar_prefetch=2,
        grid=(B // NB,),
        in_specs=[
            pl.BlockSpec(memory_space=pltpu.SMEM),
        ] + row_specs * 4 + [
            pl.BlockSpec((NB, 3, 512, 512),
                         lambda bi, idr, dtr: (bi, 0, 0, 0)),
        ],
        out_specs=pl.BlockSpec((NB, 3, 512, 512),
                               lambda bi, idr, dtr: (bi, 0, 0, 0)),
    )
    wi1 = wident1.reshape(-1, 1, 3)
    bi1 = bident1.reshape(-1, 1, 3)
    wi2 = wident2.reshape(-1, 1, 3)
    bi2 = bident2.reshape(-1, 1, 3)
    return pl.pallas_call(
        _affine_body,
        grid_spec=grid_spec,
        out_shape=jax.ShapeDtypeStruct(image.shape, image.dtype),
        compiler_params=pltpu.CompilerParams(
            dimension_semantics=("arbitrary",)),
    )(idindex, dataset_type, wbcam,
      *([wi1] * NB), *([bi1] * NB), *([wi2] * NB), *([bi2] * NB), image)


@jax.jit
def kernel(image, camindex, idindex, dataset_type,
           wcam1, bcam1, wident1, bident1,
           wcam2, bcam2, wident2, bident2):
    use1 = (dataset_type == 0)[:, None]
    wcam = jnp.where(use1, jnp.take(wcam1, camindex, axis=0),
                     jnp.take(wcam2, camindex, axis=0))
    bcam = jnp.where(use1, jnp.take(bcam1, camindex, axis=0),
                     jnp.take(bcam2, camindex, axis=0))
    wbcam = jnp.concatenate([wcam.T, bcam.T])
    return _tc_affine(idindex, dataset_type, wbcam,
                      wident1, bident1, wident2, bident2, image)


# speedup vs baseline: 1.3027x; 1.0068x over previous
import jax
import jax.numpy as jnp
from jax.experimental import pallas as pl
from jax.experimental.pallas import tpu as pltpu

B = 16
NB = 4


def _body(idr_ref, dtr_ref, cam_ref,
          wc1_ref, bc1_ref, wc2_ref, bc2_ref,
          *refs):
    wi1_refs = refs[0 * NB:1 * NB]
    bi1_refs = refs[1 * NB:2 * NB]
    wi2_refs = refs[2 * NB:3 * NB]
    bi2_refs = refs[3 * NB:4 * NB]
    img_ref = refs[4 * NB]
    out_ref = refs[4 * NB + 1]
    b_i = pl.program_id(0)
    for j in range(NB):
        s = b_i * NB + j
        cam = cam_ref[s]
        use1 = dtr_ref[s] == 0
        wc1 = wc1_ref[pl.ds(cam, 1), :]
        bc1 = bc1_ref[pl.ds(cam, 1), :]
        wc2 = wc2_ref[pl.ds(cam, 1), :]
        bc2 = bc2_ref[pl.ds(cam, 1), :]
        w = jnp.where(use1, wc1 + wi1_refs[j][0], wc2 + wi2_refs[j][0])
        bb = jnp.where(use1, bc1 + bi1_refs[j][0], bc2 + bi2_refs[j][0])
        for c in range(3):
            out_ref[j, c] = (img_ref[j, c] * w[0:1, c:c + 1]
                             + bb[0:1, c:c + 1])


@jax.jit
def kernel(image, camindex, idindex, dataset_type,
           wcam1, bcam1, wident1, bident1,
           wcam2, bcam2, wident2, bident2):
    def row_map(j):
        return lambda bi, idr, dtr: (idr[bi * NB + j], 0, 0)

    def full(shape):
        return pl.BlockSpec(shape, lambda bi, idr, dtr: (0, 0))

    row_specs = [pl.BlockSpec((1, 1, 3), row_map(j)) for j in range(NB)]
    grid_spec = pltpu.PrefetchScalarGridSpec(
        num_scalar_prefetch=2,
        grid=(B // NB,),
        in_specs=[
            pl.BlockSpec(memory_space=pltpu.SMEM),
            full(wcam1.shape), full(bcam1.shape),
            full(wcam2.shape), full(bcam2.shape),
        ] + row_specs * 4 + [
            pl.BlockSpec((NB, 3, 512, 512),
                         lambda bi, idr, dtr: (bi, 0, 0, 0)),
        ],
        out_specs=pl.BlockSpec((NB, 3, 512, 512),
                               lambda bi, idr, dtr: (bi, 0, 0, 0)),
    )
    wi1 = wident1.reshape(-1, 1, 3)
    bi1 = bident1.reshape(-1, 1, 3)
    wi2 = wident2.reshape(-1, 1, 3)
    bi2 = bident2.reshape(-1, 1, 3)
    return pl.pallas_call(
        _body,
        grid_spec=grid_spec,
        out_shape=jax.ShapeDtypeStruct(image.shape, image.dtype),
        compiler_params=pltpu.CompilerParams(
            dimension_semantics=("arbitrary",)),
    )(idindex, dataset_type, camindex,
      wcam1, bcam1, wcam2, bcam2,
      *([wi1] * NB), *([bi1] * NB), *([wi2] * NB), *([bi2] * NB), image)
